# baseline (speedup 1.0000x reference)
import functools

import jax
import jax.numpy as jnp
from jax import lax
from jax.experimental import pallas as pl
from jax.experimental.pallas import tpu as pltpu
from jax.experimental.pallas import tpu_sc as plsc

NC = 2
NS = 16
L = 16
NW = NC * NS


def _glove_body(E, PW, NCHUNK, NG,
                cidx_h, oidx_h, co_h, wt_h, wc_h, wo_h, bc_h, bo_h,
                out_h,
                idxc_v, idxo_v, rc_v, ro_v, bcv, bov, cov, wtv, accv, sem):
    wid = lax.axis_index("s") * NC + lax.axis_index("c")

    pltpu.sync_copy(cidx_h.at[wid], idxc_v)
    pltpu.sync_copy(oidx_h.at[wid], idxo_v)

    copies = []
    for k in range(NCHUNK):
        copies.append(pltpu.async_copy(
            wc_h.at[idxc_v.at[k]], rc_v.at[pl.ds(k * 128, 128)], sem))
        copies.append(pltpu.async_copy(
            wo_h.at[idxo_v.at[k]], ro_v.at[pl.ds(k * 128, 128)], sem))
        copies.append(pltpu.async_copy(
            bc_h.at[idxc_v.at[k]], bcv.at[pl.ds(k * 128, 128)], sem))
        copies.append(pltpu.async_copy(
            bo_h.at[idxo_v.at[k]], bov.at[pl.ds(k * 128, 128)], sem))
    pltpu.sync_copy(co_h.at[wid], cov)
    pltpu.sync_copy(wt_h.at[wid], wtv)
    for c in copies:
        c.wait()

    iota = lax.iota(jnp.int32, L)
    jvs = [jnp.full((L,), j, jnp.int32) for j in range(E)]

    def group(g, acc):
        riv = iota + g * L
        dot = jnp.zeros((L,), jnp.float32)
        for j in range(E):
            cv = plsc.load_gather(rc_v, [riv, jvs[j]])
            ov = plsc.load_gather(ro_v, [riv, jvs[j]])
            dot = dot + cv * ov
        bcg = bcv[pl.ds(g * L, L)]
        bog = bov[pl.ds(g * L, L)]
        cog = cov[pl.ds(g * L, L)]
        wtg = wtv[pl.ds(g * L, L)]
        err = dot + bcg + bog - cog
        return acc + wtg * err * err

    acc = lax.fori_loop(0, NG, group, jnp.zeros((L,), jnp.float32))
    accv[...] = acc
    pltpu.sync_copy(accv, out_h.at[wid])


def kernel(center, outside, coocs, weighting, W_center, W_outside,
           b_center, b_outside):
    B = center.shape[0]
    E = W_center.shape[1]
    PW = B // NW
    NCHUNK = PW // 128
    NG = PW // L

    cidx = center.reshape(NW, NCHUNK, 128).astype(jnp.int32)
    oidx = outside.reshape(NW, NCHUNK, 128).astype(jnp.int32)
    co2 = coocs.reshape(NW, PW)
    wt2 = weighting.reshape(NW, PW)

    mesh = plsc.VectorSubcoreMesh(core_axis_name="c", subcore_axis_name="s",
                                  num_cores=NC, num_subcores=NS)
    body = functools.partial(_glove_body, E, PW, NCHUNK, NG)
    partials = pl.kernel(
        body,
        out_type=jax.ShapeDtypeStruct((NW, L), jnp.float32),
        mesh=mesh,
        compiler_params=pltpu.CompilerParams(needs_layout_passes=False, use_tc_tiling_on_sc=False),
        scratch_types=[
            pltpu.VMEM((NCHUNK, 128), jnp.int32),
            pltpu.VMEM((NCHUNK, 128), jnp.int32),
            pltpu.VMEM((PW, E), jnp.float32),
            pltpu.VMEM((PW, E), jnp.float32),
            pltpu.VMEM((PW,), jnp.float32),
            pltpu.VMEM((PW,), jnp.float32),
            pltpu.VMEM((PW,), jnp.float32),
            pltpu.VMEM((PW,), jnp.float32),
            pltpu.VMEM((L,), jnp.float32),
            pltpu.SemaphoreType.DMA,
        ],
    )(cidx, oidx, co2, wt2, W_center, W_outside,
      b_center.reshape(-1), b_outside.reshape(-1))
    return jnp.sum(partials)


# speedup vs baseline: 1.2212x; 1.2212x over previous
import functools

import jax
import jax.numpy as jnp
from jax import lax
from jax.experimental import pallas as pl
from jax.experimental.pallas import tpu as pltpu
from jax.experimental.pallas import tpu_sc as plsc

NC = 2
NS = 16
L = 16
NW = NC * NS


def _glove_body(E, PW, NG,
                cidx_h, oidx_h, co_h, wt_h, wc_h, wo_h, bc_h, bo_h,
                out_h, *refs):
    ctb = refs[0:L]
    otb = refs[L:2 * L]
    (cix_v, oix_v, bcv, bov, cov, wtv, accv, sem, bsem) = refs[2 * L:]

    wid = lax.axis_index("s") * NC + lax.axis_index("c")
    base = wid * PW

    pltpu.sync_copy(cidx_h.at[pl.ds(base, PW)], cix_v)
    pltpu.sync_copy(oidx_h.at[pl.ds(base, PW)], oix_v)

    bias_copies = []
    for k in range(PW // 128):
        bias_copies.append(pltpu.async_copy(
            bc_h.at[cix_v.at[pl.ds(k * 128, 128)]],
            bcv.at[pl.ds(k * 128, 128)], bsem))
        bias_copies.append(pltpu.async_copy(
            bo_h.at[oix_v.at[pl.ds(k * 128, 128)]],
            bov.at[pl.ds(k * 128, 128)], bsem))
    pltpu.sync_copy(co_h.at[pl.ds(base, PW)], cov)
    pltpu.sync_copy(wt_h.at[pl.ds(base, PW)], wtv)

    def group(g, acc):
        cvv = cix_v[pl.ds(g * L, L)]
        ovv = oix_v[pl.ds(g * L, L)]
        copies = []
        for l in range(L):
            tc = cvv[l] >> 3
            to = ovv[l] >> 3
            copies.append(pltpu.async_copy(
                wc_h.at[pl.ds(tc * 8, 8)], ctb[l], sem))
            copies.append(pltpu.async_copy(
                wo_h.at[pl.ds(to * 8, 8)], otb[l], sem))
        bcg = bcv[pl.ds(g * L, L)]
        bog = bov[pl.ds(g * L, L)]
        cog = cov[pl.ds(g * L, L)]
        wtg = wtv[pl.ds(g * L, L)]
        errv = bcg + bog - cog
        for c in copies:
            c.wait()
        for l in range(L):
            sc = cvv[l] & 7
            so = ovv[l] & 7
            c0 = ctb[l][sc, pl.ds(0, L)]
            c1 = ctb[l][sc, pl.ds(L, L)]
            o0 = otb[l][so, pl.ds(0, L)]
            o1 = otb[l][so, pl.ds(L, L)]
            dot = jnp.sum(c0 * o0 + c1 * o1)
            err = dot + errv[l]
            acc = acc + wtg[l] * err * err
        return acc

    acc = lax.fori_loop(0, NG, group, jnp.float32(0))
    for bcp in bias_copies:
        bcp.wait()
    accv[...] = jnp.zeros((L,), jnp.float32) + acc
    pltpu.sync_copy(accv, out_h.at[pl.ds(wid * L, L)])


def kernel(center, outside, coocs, weighting, W_center, W_outside,
           b_center, b_outside):
    B = center.shape[0]
    V, E = W_center.shape
    PW = B // NW
    NG = PW // L

    cflat = center.reshape(-1).astype(jnp.int32)
    oflat = outside.reshape(-1).astype(jnp.int32)
    co1 = coocs.reshape(-1)
    wt1 = weighting.reshape(-1)

    mesh = plsc.VectorSubcoreMesh(core_axis_name="c", subcore_axis_name="s",
                                  num_cores=NC, num_subcores=NS)
    body = functools.partial(_glove_body, E, PW, NG)
    tile_bufs = [pltpu.VMEM((8, E), jnp.float32) for _ in range(2 * L)]
    partials = pl.kernel(
        body,
        out_type=jax.ShapeDtypeStruct((NW * L,), jnp.float32),
        mesh=mesh,
        compiler_params=pltpu.CompilerParams(
            needs_layout_passes=False, use_tc_tiling_on_sc=True),
        scratch_types=tile_bufs + [
            pltpu.VMEM((PW,), jnp.int32),
            pltpu.VMEM((PW,), jnp.int32),
            pltpu.VMEM((PW,), jnp.float32),
            pltpu.VMEM((PW,), jnp.float32),
            pltpu.VMEM((PW,), jnp.float32),
            pltpu.VMEM((PW,), jnp.float32),
            pltpu.VMEM((L,), jnp.float32),
            pltpu.SemaphoreType.DMA,
            pltpu.SemaphoreType.DMA,
        ],
    )(cflat, oflat, co1, wt1, W_center, W_outside,
      b_center.reshape(-1), b_outside.reshape(-1))
    return jnp.sum(partials.reshape(NW, L)[:, 0])


# speedup vs baseline: 1.2924x; 1.0583x over previous
import functools

import jax
import jax.numpy as jnp
from jax import lax
from jax.experimental import pallas as pl
from jax.experimental.pallas import tpu as pltpu
from jax.experimental.pallas import tpu_sc as plsc

NC = 2
NS = 16
L = 16
NW = NC * NS


def _glove_body(E, PW, NG,
                cidx_h, oidx_h, co_h, wt_h, wc_h, wo_h, bc_h, bo_h,
                out_h, *refs):
    ctb = refs[0:1]
    otb = refs[1:2]
    (cix_v, oix_v, bcv, bov, cov, wtv, accv, sem, bsem) = refs[2:]

    wid = lax.axis_index("s") * NC + lax.axis_index("c")
    base = wid * PW

    pltpu.sync_copy(cidx_h.at[pl.ds(base, PW)], cix_v)
    pltpu.sync_copy(oidx_h.at[pl.ds(base, PW)], oix_v)

    bias_copies = []
    for k in range(PW // 128):
        bias_copies.append(pltpu.async_copy(
            bc_h.at[cix_v.at[pl.ds(k * 128, 128)]],
            bcv.at[pl.ds(k * 128, 128)], bsem))
        bias_copies.append(pltpu.async_copy(
            bo_h.at[oix_v.at[pl.ds(k * 128, 128)]],
            bov.at[pl.ds(k * 128, 128)], bsem))
    pltpu.sync_copy(co_h.at[pl.ds(base, PW)], cov)
    pltpu.sync_copy(wt_h.at[pl.ds(base, PW)], wtv)

    def group(g, acc):
        cvv = cix_v[pl.ds(g * L, L)]
        ovv = oix_v[pl.ds(g * L, L)]
        copies = []
        for l in range(L):
            tc = cvv[l]
            to = ovv[l]
            copies.append(pltpu.async_copy(
                wc_h.at[pl.ds(tc, 1)], ctb[0].at[pl.ds(l, 1)], sem))
            copies.append(pltpu.async_copy(
                wo_h.at[pl.ds(to, 1)], otb[0].at[pl.ds(l, 1)], sem))
        bcg = bcv[pl.ds(g * L, L)]
        bog = bov[pl.ds(g * L, L)]
        cog = cov[pl.ds(g * L, L)]
        wtg = wtv[pl.ds(g * L, L)]
        errv = bcg + bog - cog
        for c in copies:
            c.wait()
        for l in range(L):
            c0 = ctb[0][l, pl.ds(0, L)]
            c1 = ctb[0][l, pl.ds(L, L)]
            o0 = otb[0][l, pl.ds(0, L)]
            o1 = otb[0][l, pl.ds(L, L)]
            dot = jnp.sum(c0 * o0 + c1 * o1)
            err = dot + errv[l]
            acc = acc + wtg[l] * err * err
        return acc

    acc = lax.fori_loop(0, NG, group, jnp.float32(0))
    for bcp in bias_copies:
        bcp.wait()
    accv[...] = jnp.zeros((L,), jnp.float32) + acc
    pltpu.sync_copy(accv, out_h.at[pl.ds(wid * L, L)])


def kernel(center, outside, coocs, weighting, W_center, W_outside,
           b_center, b_outside):
    B = center.shape[0]
    V, E = W_center.shape
    PW = B // NW
    NG = PW // L

    cflat = center.reshape(-1).astype(jnp.int32)
    oflat = outside.reshape(-1).astype(jnp.int32)
    co1 = coocs.reshape(-1)
    wt1 = weighting.reshape(-1)

    mesh = plsc.VectorSubcoreMesh(core_axis_name="c", subcore_axis_name="s",
                                  num_cores=NC, num_subcores=NS)
    body = functools.partial(_glove_body, E, PW, NG)
    tile_bufs = [pltpu.VMEM((L, E), jnp.float32) for _ in range(2)]
    partials = pl.kernel(
        body,
        out_type=jax.ShapeDtypeStruct((NW * L,), jnp.float32),
        mesh=mesh,
        compiler_params=pltpu.CompilerParams(
            needs_layout_passes=False, use_tc_tiling_on_sc=True),
        scratch_types=tile_bufs + [
            pltpu.VMEM((PW,), jnp.int32),
            pltpu.VMEM((PW,), jnp.int32),
            pltpu.VMEM((PW,), jnp.float32),
            pltpu.VMEM((PW,), jnp.float32),
            pltpu.VMEM((PW,), jnp.float32),
            pltpu.VMEM((PW,), jnp.float32),
            pltpu.VMEM((L,), jnp.float32),
            pltpu.SemaphoreType.DMA,
            pltpu.SemaphoreType.DMA,
        ],
    )(cflat, oflat, co1, wt1, W_center, W_outside,
      b_center.reshape(-1), b_outside.reshape(-1))
    return jnp.sum(partials.reshape(NW, L)[:, 0])
